# baseline (speedup 1.0000x reference)
import functools

import jax
import jax.numpy as jnp
from jax import lax
from jax.experimental import pallas as pl
from jax.experimental.pallas import tpu as pltpu
from jax.experimental.pallas import tpu_sc as plsc

N = 10000
K = 16
C = 128
NG = 32
NO = 64
FE = 16
KR = 0.2
B = 400
BK = B * K
GRID = N // B
S = 5
SN = N // S
SE = SN * K
SGRID = SN // B

_NC = 2
_NS = 16
_NW = _NC * _NS
_CHUNK = 128


def _sc_gather(table, idx, d):
    n_idx = idx.shape[0]
    n_chunks = n_idx // _CHUNK
    nfull = n_chunks // _NW
    extra = n_chunks - nfull * _NW
    mesh = plsc.VectorSubcoreMesh(core_axis_name="c", subcore_axis_name="s")

    @functools.partial(
        pl.kernel,
        mesh=mesh,
        out_type=jax.ShapeDtypeStruct((n_idx, d), jnp.float32),
        scratch_types=[
            pltpu.VMEM(((nfull + 1) * _CHUNK,), jnp.int32),
            pltpu.VMEM((_CHUNK, d), jnp.float32),
            pltpu.VMEM((_CHUNK, d), jnp.float32),
            pltpu.SemaphoreType.DMA,
            pltpu.SemaphoreType.DMA,
        ],
        compiler_params=pltpu.CompilerParams(use_tc_tiling_on_sc=(d % 128 == 0)),
    )
    def gather_kernel(table_hbm, idx_hbm, out_hbm, idx_v, buf0, buf1, g0, g1):
        wid = lax.axis_index("s") * _NC + lax.axis_index("c")
        start = wid * nfull + jnp.minimum(wid, extra)
        base = start * _CHUNK

        pltpu.sync_copy(idx_hbm.at[pl.ds(base, nfull * _CHUNK)],
                        idx_v.at[pl.ds(0, nfull * _CHUNK)])

        @pl.when(wid < extra)
        def _():
            pltpu.sync_copy(
                idx_hbm.at[pl.ds(base + nfull * _CHUNK, _CHUNK)],
                idx_v.at[pl.ds(nfull * _CHUNK, _CHUNK)])

        def start_gather(j, buf, sem):
            return pltpu.async_copy(
                table_hbm.at[idx_v.at[pl.ds(j * _CHUNK, _CHUNK)]], buf, sem)

        def writeback(j, buf):
            pltpu.sync_copy(buf, out_hbm.at[pl.ds(base + j * _CHUNK, _CHUNK)])

        start_gather(0, buf0, g0)

        def body(k, carry):
            ja = 2 * k + 1
            jb = 2 * k + 2
            start_gather(ja, buf1, g1)
            pltpu.make_async_copy(table_hbm, buf0, g0).wait()
            writeback(ja - 1, buf0)
            start_gather(jb, buf0, g0)
            pltpu.make_async_copy(table_hbm, buf1, g1).wait()
            writeback(ja, buf1)
            return carry

        lax.fori_loop(0, (nfull - 1) // 2, body, 0)

        @pl.when(wid < extra)
        def _():
            start_gather(nfull, buf1, g1)

        pltpu.make_async_copy(table_hbm, buf0, g0).wait()
        writeback(nfull - 1, buf0)

        @pl.when(wid < extra)
        def _():
            pltpu.make_async_copy(table_hbm, buf1, g1).wait()
            writeback(nfull, buf1)

    return gather_kernel(table, idx)


def _full(shape):
    return pl.BlockSpec(shape, lambda i: (0, 0))


def _tc1(s0, coord16, cn, feat, dirs16, sig2inv, cvx_smT, W_enc, b_enc, W1,
         b1, W2, b2):
    kfac = 1.0 / (2.0 * KR * KR)

    def body(coord_r, cn_r, feat_r, dirs_r, s2i_r, cvx_r, we_r, be_r, w1_r,
             bb1_r, w2_r, bb2_r, rel_o, x_o, st_o):
        i = pl.program_id(0)
        cb = jnp.broadcast_to(coord_r[...][:, None, :], (B, K, 16))
        rel = cn_r[...].reshape(B, K, 16) - cb
        rel = rel.reshape(BK, 16)
        rel_o[...] = rel
        d2 = jnp.sum(rel * rel, axis=1, keepdims=True)
        proj = jnp.dot(rel, dirs_r[...], preferred_element_type=jnp.float32)
        resp = jnp.exp(-(proj * proj) * s2i_r[...]) * jnp.exp(-d2 * kfac)
        gib = jnp.mean(resp.reshape(B, K, NG), axis=1)
        obs = jnp.dot(gib, cvx_r[...], preferred_element_type=jnp.float32)
        feat = feat_r[...]
        fenc = jnp.dot(feat, we_r[...], preferred_element_type=jnp.float32) + be_r[...]
        g = jnp.concatenate([fenc, obs], axis=1)
        h = jax.nn.gelu(jnp.dot(g, w1_r[...], preferred_element_type=jnp.float32) + bb1_r[...])
        g2 = jnp.dot(h, w2_r[...], preferred_element_type=jnp.float32) + bb2_r[...]
        x = feat + g2
        x_o[...] = x

        @pl.when(i == 0)
        def _():
            st_o[...] = jnp.zeros((8, C), jnp.float32)

        upd = jnp.concatenate(
            [jnp.sum(x, axis=0, keepdims=True),
             jnp.sum(x * x, axis=0, keepdims=True),
             jnp.zeros((6, C), jnp.float32)], axis=0)
        st_o[...] += upd

    return pl.pallas_call(
        body,
        grid=(SGRID,),
        in_specs=[
            pl.BlockSpec((B, 16), lambda i: (s0 * SGRID + i, 0)),
            pl.BlockSpec((BK, 16), lambda i: (i, 0)),
            pl.BlockSpec((B, C), lambda i: (s0 * SGRID + i, 0)),
            _full((16, NG)), _full((1, NG)), _full((NG, NO)),
            _full((C, FE)), _full((1, FE)),
            _full((FE + NO, FE + NO)), _full((1, FE + NO)),
            _full((FE + NO, C)), _full((1, C)),
        ],
        out_specs=[
            pl.BlockSpec((BK, 16), lambda i: (i, 0)),
            pl.BlockSpec((B, C), lambda i: (i, 0)),
            _full((8, C)),
        ],
        out_shape=[
            jax.ShapeDtypeStruct((SE, 16), jnp.float32),
            jax.ShapeDtypeStruct((SN, C), jnp.float32),
            jax.ShapeDtypeStruct((8, C), jnp.float32),
        ],
        compiler_params=pltpu.CompilerParams(
            dimension_semantics=("arbitrary",)),
    )(coord16, cn, feat, dirs16, sig2inv, cvx_smT, W_enc, b_enc, W1, b1,
      W2, b2)


def _tc2(x, sc1, sh1, Wl1, bl1, WqA, WkA, ba1, Wv):
    def body(x_r, sc_r, sh_r, wl_r, bl_r, wq_r, wk_r, ba_r, wv_r,
             xn_o, qa_o, kv_o):
        xn = jax.nn.gelu(x_r[...] * sc_r[...] + sh_r[...])
        xn_o[...] = xn
        y = jnp.dot(xn, wl_r[...], preferred_element_type=jnp.float32) + bl_r[...]
        qa_o[...] = jnp.dot(y, wq_r[...], preferred_element_type=jnp.float32)
        kv_o[:, :C] = jnp.dot(y, wk_r[...], preferred_element_type=jnp.float32) + ba_r[...]
        kv_o[:, C:] = jnp.dot(y, wv_r[...], preferred_element_type=jnp.float32)

    return pl.pallas_call(
        body,
        grid=(GRID,),
        in_specs=[
            pl.BlockSpec((B, C), lambda i: (i, 0)),
            _full((1, C)), _full((1, C)),
            _full((C, C)), _full((1, C)),
            _full((C, C)), _full((C, C)), _full((1, C)), _full((C, C)),
        ],
        out_specs=[
            pl.BlockSpec((B, C), lambda i: (i, 0)),
            pl.BlockSpec((B, C), lambda i: (i, 0)),
            pl.BlockSpec((B, 2 * C), lambda i: (i, 0)),
        ],
        out_shape=[
            jax.ShapeDtypeStruct((N, C), jnp.float32),
            jax.ShapeDtypeStruct((N, C), jnp.float32),
            jax.ShapeDtypeStruct((N, 2 * C), jnp.float32),
        ],
        compiler_params=pltpu.CompilerParams(
            dimension_semantics=("arbitrary",)),
    )(x, sc1, sh1, Wl1, bl1, WqA, WkA, ba1, Wv)


def _tc3(s0, kvn, rel16, qA, xn, Wp1p, bp1p, Wp2p, bp2, Wp2Ap, bp2A, Wa2,
         ba2, Wl2, bl2, Ws1, bs1, Ws2, bs2):
    def body(kvn_r, rel_r, qa_r, xn_r, wp1_r, bp1_r, wp2_r, bp2_r, wp2a_r,
             bp2a_r, wa2_r, ba2_r, wl2_r, bl2_r, ws1_r, bs1_r, ws2_r, bs2_r,
             s_o, st_o):
        i = pl.program_id(0)
        rel = rel_r[...]
        e = jax.nn.relu(jnp.dot(rel, wp1_r[...], preferred_element_type=jnp.float32) + bp1_r[...])
        pos = jnp.dot(e, wp2_r[...], preferred_element_type=jnp.float32) + bp2_r[...]
        posA = jnp.dot(e, wp2a_r[...], preferred_element_type=jnp.float32) + bp2a_r[...]
        kan = kvn_r[:, :C]
        vn = kvn_r[:, C:]
        qrep = jnp.broadcast_to(qa_r[...][:, None, :], (B, K, C)).reshape(BK, C)
        w1 = jax.nn.relu(kan - qrep + posA)
        w = jnp.dot(w1, wa2_r[...], preferred_element_type=jnp.float32) + ba2_r[...]
        w3 = w.reshape(B, K, C)
        m = jnp.max(w3, axis=1, keepdims=True)
        ew = jnp.exp(w3 - m)
        ssum = jnp.sum(ew, axis=1)
        z = (vn + pos).reshape(B, K, C)
        agg = jnp.sum(ew * z, axis=1) / ssum
        x2 = jax.nn.relu(
            xn_r[...] + jnp.dot(agg, wl2_r[...], preferred_element_type=jnp.float32) + bl2_r[...])
        h = jax.nn.gelu(jnp.dot(x2, ws1_r[...], preferred_element_type=jnp.float32) + bs1_r[...])
        s = jnp.dot(h, ws2_r[...], preferred_element_type=jnp.float32) + bs2_r[...]
        s_o[...] = s

        @pl.when(i == 0)
        def _():
            st_o[...] = jnp.zeros((8, C), jnp.float32)

        upd = jnp.concatenate(
            [jnp.sum(s, axis=0, keepdims=True),
             jnp.sum(s * s, axis=0, keepdims=True),
             jnp.zeros((6, C), jnp.float32)], axis=0)
        st_o[...] += upd

    return pl.pallas_call(
        body,
        grid=(SGRID,),
        in_specs=[
            pl.BlockSpec((BK, 2 * C), lambda i: (i, 0)),
            pl.BlockSpec((BK, 16), lambda i: (i, 0)),
            pl.BlockSpec((B, C), lambda i: (s0 * SGRID + i, 0)),
            pl.BlockSpec((B, C), lambda i: (s0 * SGRID + i, 0)),
            _full((16, 16)), _full((1, 16)),
            _full((16, C)), _full((1, C)),
            _full((16, C)), _full((1, C)),
            _full((C, C)), _full((1, C)),
            _full((C, C)), _full((1, C)),
            _full((C, C)), _full((1, C)),
            _full((C, C)), _full((1, C)),
        ],
        out_specs=[
            pl.BlockSpec((B, C), lambda i: (i, 0)),
            _full((8, C)),
        ],
        out_shape=[
            jax.ShapeDtypeStruct((SN, C), jnp.float32),
            jax.ShapeDtypeStruct((8, C), jnp.float32),
        ],
        compiler_params=pltpu.CompilerParams(
            dimension_semantics=("arbitrary",)),
    )(kvn, rel16, qA, xn, Wp1p, bp1p, Wp2p, bp2, Wp2Ap, bp2A, Wa2, ba2,
      Wl2, bl2, Ws1, bs1, Ws2, bs2)


def _tc4(s, sc2, sh2):
    B4 = 1000

    def body(s_r, sc_r, sh_r, o_r):
        o_r[...] = jax.nn.gelu(s_r[...] * sc_r[...] + sh_r[...])

    return pl.pallas_call(
        body,
        grid=(N // B4,),
        in_specs=[
            pl.BlockSpec((B4, C), lambda i: (i, 0)),
            _full((1, C)), _full((1, C)),
        ],
        out_specs=pl.BlockSpec((B4, C), lambda i: (i, 0)),
        out_shape=jax.ShapeDtypeStruct((N, C), jnp.float32),
    )(s, sc2, sh2)


def _bn_scale_shift(ssum, ssq, g, b):
    m = ssum / N
    v = ssq / N - m * m
    sc = g / jnp.sqrt(v + 1e-5)
    return sc[None, :], (b - m * sc)[None, :]


def kernel(coord, feat, offset, neighbor_idx, params):
    p = params
    dirs16 = jnp.zeros((16, NG), jnp.float32).at[:3].set(p['gib_dirs'].T)
    sig = jax.nn.softplus(p['gib_sigma']) + 1e-4
    sig2inv = (1.0 / (2.0 * sig * sig))[None, :]
    cvx_smT = jax.nn.softmax(p['cvx'], axis=1).T
    coord16 = jnp.zeros((N, 16), jnp.float32).at[:, :3].set(coord)
    WqA = p['Wq'] @ p['Wa1']
    WkA = p['Wk'] @ p['Wa1']
    Wp1p = jnp.zeros((16, 16), jnp.float32).at[:3, :3].set(p['Wp1'])
    bp1p = jnp.zeros((1, 16), jnp.float32).at[0, :3].set(p['bp1'])
    Wp2p = jnp.zeros((16, C), jnp.float32).at[:3].set(p['Wp2'])
    Wp2Ap = Wp2p @ p['Wa1']
    bp2A = (p['bp2'] @ p['Wa1'])[None, :]

    nbr_flat = neighbor_idx.reshape(-1)

    rel16_s, x_s, st1_s = [], [], []
    for s in range(S):
        cn = _sc_gather(coord16, lax.slice(nbr_flat, (s * SE,), ((s + 1) * SE,)), 16)
        rel16, x, st1 = _tc1(
            s, coord16, cn, feat, dirs16, sig2inv, cvx_smT,
            p['W_enc'], p['b_enc'][None, :], p['W1'], p['b1'][None, :],
            p['W2'], p['b2'][None, :])
        rel16_s.append(rel16)
        x_s.append(x)
        st1_s.append(st1)
    st1 = st1_s[0] + st1_s[1] + st1_s[2] + st1_s[3] + st1_s[4]
    sc1, sh1 = _bn_scale_shift(st1[0], st1[1], p['g1'], p['be1'])
    x = jnp.concatenate(x_s, axis=0)

    xn, qA, kv = _tc2(
        x, sc1, sh1, p['Wl1'], p['bl1'][None, :], WqA, WkA,
        p['ba1'][None, :], p['Wv'])

    s_s, st2_s = [], []
    for s in range(S):
        kvn = _sc_gather(kv, lax.slice(nbr_flat, (s * SE,), ((s + 1) * SE,)), 2 * C)
        ss, st2 = _tc3(
            s, kvn, rel16_s[s], qA, xn, Wp1p, bp1p, Wp2p, p['bp2'][None, :],
            Wp2Ap, bp2A, p['Wa2'], p['ba2'][None, :], p['Wl2'],
            p['bl2'][None, :], p['Ws1'], p['bs1'][None, :], p['Ws2'],
            p['bs2'][None, :])
        s_s.append(ss)
        st2_s.append(st2)
    st2 = st2_s[0] + st2_s[1] + st2_s[2] + st2_s[3] + st2_s[4]
    sc2, sh2 = _bn_scale_shift(st2[0], st2[1], p['g2'], p['be2'])
    s = jnp.concatenate(s_s, axis=0)

    out = _tc4(s, sc2, sh2)
    return (coord, out, offset)


# speedup vs baseline: 1.0514x; 1.0514x over previous
import functools

import jax
import jax.numpy as jnp
from jax import lax
from jax.experimental import pallas as pl
from jax.experimental.pallas import tpu as pltpu
from jax.experimental.pallas import tpu_sc as plsc

N = 10000
K = 16
C = 128
NG = 32
NO = 64
FE = 16
KR = 0.2
B = 400
BK = B * K
GRID = N // B
S = 1
SN = N // S
SE = SN * K
SGRID = SN // B

_NC = 2
_NS = 16
_NW = _NC * _NS
_CHUNK = 128


def _sc_gather(table, idx, d):
    n_idx = idx.shape[0]
    n_chunks = n_idx // _CHUNK
    nfull = n_chunks // _NW
    extra = n_chunks - nfull * _NW
    mesh = plsc.VectorSubcoreMesh(core_axis_name="c", subcore_axis_name="s")

    @functools.partial(
        pl.kernel,
        mesh=mesh,
        out_type=jax.ShapeDtypeStruct((n_idx, d), jnp.float32),
        scratch_types=[
            pltpu.VMEM(((nfull + 1) * _CHUNK,), jnp.int32),
            pltpu.VMEM((_CHUNK, d), jnp.float32),
            pltpu.VMEM((_CHUNK, d), jnp.float32),
            pltpu.SemaphoreType.DMA,
            pltpu.SemaphoreType.DMA,
        ],
        compiler_params=pltpu.CompilerParams(use_tc_tiling_on_sc=(d % 128 == 0)),
    )
    def gather_kernel(table_hbm, idx_hbm, out_hbm, idx_v, buf0, buf1, g0, g1):
        wid = lax.axis_index("s") * _NC + lax.axis_index("c")
        start = wid * nfull + jnp.minimum(wid, extra)
        base = start * _CHUNK

        pltpu.sync_copy(idx_hbm.at[pl.ds(base, nfull * _CHUNK)],
                        idx_v.at[pl.ds(0, nfull * _CHUNK)])

        @pl.when(wid < extra)
        def _():
            pltpu.sync_copy(
                idx_hbm.at[pl.ds(base + nfull * _CHUNK, _CHUNK)],
                idx_v.at[pl.ds(nfull * _CHUNK, _CHUNK)])

        def start_gather(j, buf, sem):
            return pltpu.async_copy(
                table_hbm.at[idx_v.at[pl.ds(j * _CHUNK, _CHUNK)]], buf, sem)

        def writeback(j, buf):
            pltpu.sync_copy(buf, out_hbm.at[pl.ds(base + j * _CHUNK, _CHUNK)])

        start_gather(0, buf0, g0)

        def body(k, carry):
            ja = 2 * k + 1
            jb = 2 * k + 2
            start_gather(ja, buf1, g1)
            pltpu.make_async_copy(table_hbm, buf0, g0).wait()
            writeback(ja - 1, buf0)
            start_gather(jb, buf0, g0)
            pltpu.make_async_copy(table_hbm, buf1, g1).wait()
            writeback(ja, buf1)
            return carry

        lax.fori_loop(0, (nfull - 1) // 2, body, 0)

        @pl.when(wid < extra)
        def _():
            start_gather(nfull, buf1, g1)

        pltpu.make_async_copy(table_hbm, buf0, g0).wait()
        writeback(nfull - 1, buf0)

        @pl.when(wid < extra)
        def _():
            pltpu.make_async_copy(table_hbm, buf1, g1).wait()
            writeback(nfull, buf1)

    return gather_kernel(table, idx)


def _full(shape):
    return pl.BlockSpec(shape, lambda i: (0, 0))


def _tc1(s0, coord16, cn, feat, dirs16, sig2inv, cvx_smT, W_enc, b_enc, W1,
         b1, W2, b2):
    kfac = 1.0 / (2.0 * KR * KR)

    def body(coord_r, cn_r, feat_r, dirs_r, s2i_r, cvx_r, we_r, be_r, w1_r,
             bb1_r, w2_r, bb2_r, rel_o, x_o, st_o):
        i = pl.program_id(0)
        cb = jnp.broadcast_to(coord_r[...][:, None, :], (B, K, 16))
        rel = cn_r[...].reshape(B, K, 16) - cb
        rel = rel.reshape(BK, 16)
        rel_o[...] = rel
        d2 = jnp.sum(rel * rel, axis=1, keepdims=True)
        proj = jnp.dot(rel, dirs_r[...], preferred_element_type=jnp.float32)
        resp = jnp.exp(-(proj * proj) * s2i_r[...]) * jnp.exp(-d2 * kfac)
        gib = jnp.mean(resp.reshape(B, K, NG), axis=1)
        obs = jnp.dot(gib, cvx_r[...], preferred_element_type=jnp.float32)
        feat = feat_r[...]
        fenc = jnp.dot(feat, we_r[...], preferred_element_type=jnp.float32) + be_r[...]
        g = jnp.concatenate([fenc, obs], axis=1)
        h = jax.nn.gelu(jnp.dot(g, w1_r[...], preferred_element_type=jnp.float32) + bb1_r[...])
        g2 = jnp.dot(h, w2_r[...], preferred_element_type=jnp.float32) + bb2_r[...]
        x = feat + g2
        x_o[...] = x

        @pl.when(i == 0)
        def _():
            st_o[...] = jnp.zeros((8, C), jnp.float32)

        upd = jnp.concatenate(
            [jnp.sum(x, axis=0, keepdims=True),
             jnp.sum(x * x, axis=0, keepdims=True),
             jnp.zeros((6, C), jnp.float32)], axis=0)
        st_o[...] += upd

    return pl.pallas_call(
        body,
        grid=(SGRID,),
        in_specs=[
            pl.BlockSpec((B, 16), lambda i: (s0 * SGRID + i, 0)),
            pl.BlockSpec((BK, 16), lambda i: (i, 0)),
            pl.BlockSpec((B, C), lambda i: (s0 * SGRID + i, 0)),
            _full((16, NG)), _full((1, NG)), _full((NG, NO)),
            _full((C, FE)), _full((1, FE)),
            _full((FE + NO, FE + NO)), _full((1, FE + NO)),
            _full((FE + NO, C)), _full((1, C)),
        ],
        out_specs=[
            pl.BlockSpec((BK, 16), lambda i: (i, 0)),
            pl.BlockSpec((B, C), lambda i: (i, 0)),
            _full((8, C)),
        ],
        out_shape=[
            jax.ShapeDtypeStruct((SE, 16), jnp.float32),
            jax.ShapeDtypeStruct((SN, C), jnp.float32),
            jax.ShapeDtypeStruct((8, C), jnp.float32),
        ],
        compiler_params=pltpu.CompilerParams(
            dimension_semantics=("arbitrary",)),
    )(coord16, cn, feat, dirs16, sig2inv, cvx_smT, W_enc, b_enc, W1, b1,
      W2, b2)


def _tc2(x, sc1, sh1, Wl1, bl1, WqA, WkA, ba1, Wv):
    def body(x_r, sc_r, sh_r, wl_r, bl_r, wq_r, wk_r, ba_r, wv_r,
             xn_o, qa_o, kv_o):
        xn = jax.nn.gelu(x_r[...] * sc_r[...] + sh_r[...])
        xn_o[...] = xn
        y = jnp.dot(xn, wl_r[...], preferred_element_type=jnp.float32) + bl_r[...]
        qa_o[...] = jnp.dot(y, wq_r[...], preferred_element_type=jnp.float32)
        kv_o[:, :C] = jnp.dot(y, wk_r[...], preferred_element_type=jnp.float32) + ba_r[...]
        kv_o[:, C:] = jnp.dot(y, wv_r[...], preferred_element_type=jnp.float32)

    return pl.pallas_call(
        body,
        grid=(GRID,),
        in_specs=[
            pl.BlockSpec((B, C), lambda i: (i, 0)),
            _full((1, C)), _full((1, C)),
            _full((C, C)), _full((1, C)),
            _full((C, C)), _full((C, C)), _full((1, C)), _full((C, C)),
        ],
        out_specs=[
            pl.BlockSpec((B, C), lambda i: (i, 0)),
            pl.BlockSpec((B, C), lambda i: (i, 0)),
            pl.BlockSpec((B, 2 * C), lambda i: (i, 0)),
        ],
        out_shape=[
            jax.ShapeDtypeStruct((N, C), jnp.float32),
            jax.ShapeDtypeStruct((N, C), jnp.float32),
            jax.ShapeDtypeStruct((N, 2 * C), jnp.float32),
        ],
        compiler_params=pltpu.CompilerParams(
            dimension_semantics=("arbitrary",)),
    )(x, sc1, sh1, Wl1, bl1, WqA, WkA, ba1, Wv)


def _tc3(s0, kvn, rel16, qA, xn, Wp1p, bp1p, Wp2p, bp2, Wp2Ap, bp2A, Wa2,
         ba2, Wl2, bl2, Ws1, bs1, Ws2, bs2):
    def body(kvn_r, rel_r, qa_r, xn_r, wp1_r, bp1_r, wp2_r, bp2_r, wp2a_r,
             bp2a_r, wa2_r, ba2_r, wl2_r, bl2_r, ws1_r, bs1_r, ws2_r, bs2_r,
             s_o, st_o):
        i = pl.program_id(0)
        rel = rel_r[...]
        e = jax.nn.relu(jnp.dot(rel, wp1_r[...], preferred_element_type=jnp.float32) + bp1_r[...])
        pos = jnp.dot(e, wp2_r[...], preferred_element_type=jnp.float32) + bp2_r[...]
        posA = jnp.dot(e, wp2a_r[...], preferred_element_type=jnp.float32) + bp2a_r[...]
        kan = kvn_r[:, :C]
        vn = kvn_r[:, C:]
        qrep = jnp.broadcast_to(qa_r[...][:, None, :], (B, K, C)).reshape(BK, C)
        w1 = jax.nn.relu(kan - qrep + posA)
        w = jnp.dot(w1, wa2_r[...], preferred_element_type=jnp.float32) + ba2_r[...]
        w3 = w.reshape(B, K, C)
        m = jnp.max(w3, axis=1, keepdims=True)
        ew = jnp.exp(w3 - m)
        ssum = jnp.sum(ew, axis=1)
        z = (vn + pos).reshape(B, K, C)
        agg = jnp.sum(ew * z, axis=1) / ssum
        x2 = jax.nn.relu(
            xn_r[...] + jnp.dot(agg, wl2_r[...], preferred_element_type=jnp.float32) + bl2_r[...])
        h = jax.nn.gelu(jnp.dot(x2, ws1_r[...], preferred_element_type=jnp.float32) + bs1_r[...])
        s = jnp.dot(h, ws2_r[...], preferred_element_type=jnp.float32) + bs2_r[...]
        s_o[...] = s

        @pl.when(i == 0)
        def _():
            st_o[...] = jnp.zeros((8, C), jnp.float32)

        upd = jnp.concatenate(
            [jnp.sum(s, axis=0, keepdims=True),
             jnp.sum(s * s, axis=0, keepdims=True),
             jnp.zeros((6, C), jnp.float32)], axis=0)
        st_o[...] += upd

    return pl.pallas_call(
        body,
        grid=(SGRID,),
        in_specs=[
            pl.BlockSpec((BK, 2 * C), lambda i: (i, 0)),
            pl.BlockSpec((BK, 16), lambda i: (i, 0)),
            pl.BlockSpec((B, C), lambda i: (s0 * SGRID + i, 0)),
            pl.BlockSpec((B, C), lambda i: (s0 * SGRID + i, 0)),
            _full((16, 16)), _full((1, 16)),
            _full((16, C)), _full((1, C)),
            _full((16, C)), _full((1, C)),
            _full((C, C)), _full((1, C)),
            _full((C, C)), _full((1, C)),
            _full((C, C)), _full((1, C)),
            _full((C, C)), _full((1, C)),
        ],
        out_specs=[
            pl.BlockSpec((B, C), lambda i: (i, 0)),
            _full((8, C)),
        ],
        out_shape=[
            jax.ShapeDtypeStruct((SN, C), jnp.float32),
            jax.ShapeDtypeStruct((8, C), jnp.float32),
        ],
        compiler_params=pltpu.CompilerParams(
            dimension_semantics=("arbitrary",)),
    )(kvn, rel16, qA, xn, Wp1p, bp1p, Wp2p, bp2, Wp2Ap, bp2A, Wa2, ba2,
      Wl2, bl2, Ws1, bs1, Ws2, bs2)


def _tc4(s, sc2, sh2):
    B4 = 1000

    def body(s_r, sc_r, sh_r, o_r):
        o_r[...] = jax.nn.gelu(s_r[...] * sc_r[...] + sh_r[...])

    return pl.pallas_call(
        body,
        grid=(N // B4,),
        in_specs=[
            pl.BlockSpec((B4, C), lambda i: (i, 0)),
            _full((1, C)), _full((1, C)),
        ],
        out_specs=pl.BlockSpec((B4, C), lambda i: (i, 0)),
        out_shape=jax.ShapeDtypeStruct((N, C), jnp.float32),
    )(s, sc2, sh2)


def _bn_scale_shift(ssum, ssq, g, b):
    m = ssum / N
    v = ssq / N - m * m
    sc = g / jnp.sqrt(v + 1e-5)
    return sc[None, :], (b - m * sc)[None, :]


def kernel(coord, feat, offset, neighbor_idx, params):
    p = params
    dirs16 = jnp.zeros((16, NG), jnp.float32).at[:3].set(p['gib_dirs'].T)
    sig = jax.nn.softplus(p['gib_sigma']) + 1e-4
    sig2inv = (1.0 / (2.0 * sig * sig))[None, :]
    cvx_smT = jax.nn.softmax(p['cvx'], axis=1).T
    coord16 = jnp.zeros((N, 16), jnp.float32).at[:, :3].set(coord)
    WqA = p['Wq'] @ p['Wa1']
    WkA = p['Wk'] @ p['Wa1']
    Wp1p = jnp.zeros((16, 16), jnp.float32).at[:3, :3].set(p['Wp1'])
    bp1p = jnp.zeros((1, 16), jnp.float32).at[0, :3].set(p['bp1'])
    Wp2p = jnp.zeros((16, C), jnp.float32).at[:3].set(p['Wp2'])
    Wp2Ap = Wp2p @ p['Wa1']
    bp2A = (p['bp2'] @ p['Wa1'])[None, :]

    nbr_flat = neighbor_idx.reshape(-1)

    rel16_s, x_s, st1_s = [], [], []
    for s in range(S):
        cn = _sc_gather(coord16, lax.slice(nbr_flat, (s * SE,), ((s + 1) * SE,)), 16)
        rel16, x, st1 = _tc1(
            s, coord16, cn, feat, dirs16, sig2inv, cvx_smT,
            p['W_enc'], p['b_enc'][None, :], p['W1'], p['b1'][None, :],
            p['W2'], p['b2'][None, :])
        rel16_s.append(rel16)
        x_s.append(x)
        st1_s.append(st1)
    st1 = functools.reduce(lambda a, b: a + b, st1_s)
    sc1, sh1 = _bn_scale_shift(st1[0], st1[1], p['g1'], p['be1'])
    x = jnp.concatenate(x_s, axis=0)

    xn, qA, kv = _tc2(
        x, sc1, sh1, p['Wl1'], p['bl1'][None, :], WqA, WkA,
        p['ba1'][None, :], p['Wv'])

    s_s, st2_s = [], []
    for s in range(S):
        kvn = _sc_gather(kv, lax.slice(nbr_flat, (s * SE,), ((s + 1) * SE,)), 2 * C)
        ss, st2 = _tc3(
            s, kvn, rel16_s[s], qA, xn, Wp1p, bp1p, Wp2p, p['bp2'][None, :],
            Wp2Ap, bp2A, p['Wa2'], p['ba2'][None, :], p['Wl2'],
            p['bl2'][None, :], p['Ws1'], p['bs1'][None, :], p['Ws2'],
            p['bs2'][None, :])
        s_s.append(ss)
        st2_s.append(st2)
    st2 = functools.reduce(lambda a, b: a + b, st2_s)
    sc2, sh2 = _bn_scale_shift(st2[0], st2[1], p['g2'], p['be2'])
    s = jnp.concatenate(s_s, axis=0)

    out = _tc4(s, sc2, sh2)
    return (coord, out, offset)


# speedup vs baseline: 1.1890x; 1.1309x over previous
import functools

import jax
import jax.numpy as jnp
from jax import lax
from jax.experimental import pallas as pl
from jax.experimental.pallas import tpu as pltpu
from jax.experimental.pallas import tpu_sc as plsc

N = 10000
K = 16
C = 128
NG = 32
NO = 64
FE = 16
KR = 0.2
B = 400
BK = B * K
GRID = N // B
S = 1
SN = N // S
SE = SN * K
SGRID = SN // B

_NC = 2
_NS = 16
_NW = _NC * _NS
_CHUNK = 128


def _sc_gather(table, idx, d, dtype=jnp.float32):
    n_idx = idx.shape[0]
    n_chunks = n_idx // _CHUNK
    nfull = n_chunks // _NW
    extra = n_chunks - nfull * _NW
    mesh = plsc.VectorSubcoreMesh(core_axis_name="c", subcore_axis_name="s")

    @functools.partial(
        pl.kernel,
        mesh=mesh,
        out_type=jax.ShapeDtypeStruct((n_idx, d), dtype),
        scratch_types=[
            pltpu.VMEM(((nfull + 1) * _CHUNK,), jnp.int32),
            pltpu.VMEM((_CHUNK, d), dtype),
            pltpu.VMEM((_CHUNK, d), dtype),
            pltpu.SemaphoreType.DMA,
            pltpu.SemaphoreType.DMA,
        ],
        compiler_params=pltpu.CompilerParams(use_tc_tiling_on_sc=(d % 128 == 0)),
    )
    def gather_kernel(table_hbm, idx_hbm, out_hbm, idx_v, buf0, buf1, g0, g1):
        wid = lax.axis_index("s") * _NC + lax.axis_index("c")
        start = wid * nfull + jnp.minimum(wid, extra)
        base = start * _CHUNK

        pltpu.sync_copy(idx_hbm.at[pl.ds(base, nfull * _CHUNK)],
                        idx_v.at[pl.ds(0, nfull * _CHUNK)])

        @pl.when(wid < extra)
        def _():
            pltpu.sync_copy(
                idx_hbm.at[pl.ds(base + nfull * _CHUNK, _CHUNK)],
                idx_v.at[pl.ds(nfull * _CHUNK, _CHUNK)])

        def start_gather(j, buf, sem):
            return pltpu.async_copy(
                table_hbm.at[idx_v.at[pl.ds(j * _CHUNK, _CHUNK)]], buf, sem)

        def writeback(j, buf):
            pltpu.sync_copy(buf, out_hbm.at[pl.ds(base + j * _CHUNK, _CHUNK)])

        start_gather(0, buf0, g0)

        def body(k, carry):
            ja = 2 * k + 1
            jb = 2 * k + 2
            start_gather(ja, buf1, g1)
            pltpu.make_async_copy(table_hbm, buf0, g0).wait()
            writeback(ja - 1, buf0)
            start_gather(jb, buf0, g0)
            pltpu.make_async_copy(table_hbm, buf1, g1).wait()
            writeback(ja, buf1)
            return carry

        lax.fori_loop(0, (nfull - 1) // 2, body, 0)

        @pl.when(wid < extra)
        def _():
            start_gather(nfull, buf1, g1)

        pltpu.make_async_copy(table_hbm, buf0, g0).wait()
        writeback(nfull - 1, buf0)

        @pl.when(wid < extra)
        def _():
            pltpu.make_async_copy(table_hbm, buf1, g1).wait()
            writeback(nfull, buf1)

    return gather_kernel(table, idx)


def _full(shape):
    return pl.BlockSpec(shape, lambda i: (0, 0))


def _tc1(s0, coord16, cn, feat, dirs16, sig2inv, cvx_smT, W_enc, b_enc, W1,
         b1, W2, b2):
    kfac = 1.0 / (2.0 * KR * KR)

    def body(coord_r, cn_r, feat_r, dirs_r, s2i_r, cvx_r, we_r, be_r, w1_r,
             bb1_r, w2_r, bb2_r, rel_o, x_o, st_o):
        i = pl.program_id(0)
        cb = jnp.broadcast_to(coord_r[...][:, None, :], (B, K, 16))
        rel = cn_r[...].reshape(B, K, 16) - cb
        rel = rel.reshape(BK, 16)
        rel_o[...] = rel
        d2 = jnp.sum(rel * rel, axis=1, keepdims=True)
        proj = jnp.dot(rel, dirs_r[...], preferred_element_type=jnp.float32)
        resp = jnp.exp(-(proj * proj) * s2i_r[...]) * jnp.exp(-d2 * kfac)
        gib = jnp.mean(resp.reshape(B, K, NG), axis=1)
        obs = jnp.dot(gib, cvx_r[...], preferred_element_type=jnp.float32)
        feat = feat_r[...]
        fenc = jnp.dot(feat, we_r[...], preferred_element_type=jnp.float32) + be_r[...]
        g = jnp.concatenate([fenc, obs], axis=1)
        h = jax.nn.gelu(jnp.dot(g, w1_r[...], preferred_element_type=jnp.float32) + bb1_r[...])
        g2 = jnp.dot(h, w2_r[...], preferred_element_type=jnp.float32) + bb2_r[...]
        x = feat + g2
        x_o[...] = x

        @pl.when(i == 0)
        def _():
            st_o[...] = jnp.zeros((8, C), jnp.float32)

        upd = jnp.concatenate(
            [jnp.sum(x, axis=0, keepdims=True),
             jnp.sum(x * x, axis=0, keepdims=True),
             jnp.zeros((6, C), jnp.float32)], axis=0)
        st_o[...] += upd

    return pl.pallas_call(
        body,
        grid=(SGRID,),
        in_specs=[
            pl.BlockSpec((B, 16), lambda i: (s0 * SGRID + i, 0)),
            pl.BlockSpec((BK, 16), lambda i: (i, 0)),
            pl.BlockSpec((B, C), lambda i: (s0 * SGRID + i, 0)),
            _full((16, NG)), _full((1, NG)), _full((NG, NO)),
            _full((C, FE)), _full((1, FE)),
            _full((FE + NO, FE + NO)), _full((1, FE + NO)),
            _full((FE + NO, C)), _full((1, C)),
        ],
        out_specs=[
            pl.BlockSpec((BK, 16), lambda i: (i, 0)),
            pl.BlockSpec((B, C), lambda i: (i, 0)),
            _full((8, C)),
        ],
        out_shape=[
            jax.ShapeDtypeStruct((SE, 16), jnp.float32),
            jax.ShapeDtypeStruct((SN, C), jnp.float32),
            jax.ShapeDtypeStruct((8, C), jnp.float32),
        ],
        compiler_params=pltpu.CompilerParams(
            dimension_semantics=("arbitrary",)),
    )(coord16, cn, feat, dirs16, sig2inv, cvx_smT, W_enc, b_enc, W1, b1,
      W2, b2)


def _tc2(x, sc1, sh1, Wl1, bl1, WqA, WkA, ba1, Wv):
    def body(x_r, sc_r, sh_r, wl_r, bl_r, wq_r, wk_r, ba_r, wv_r,
             xn_o, qa_o, kv_o):
        xn = jax.nn.gelu(x_r[...] * sc_r[...] + sh_r[...])
        xn_o[...] = xn
        y = jnp.dot(xn, wl_r[...], preferred_element_type=jnp.float32) + bl_r[...]
        qa_o[...] = jnp.dot(y, wq_r[...], preferred_element_type=jnp.float32)
        ka = jnp.dot(y, wk_r[...], preferred_element_type=jnp.float32) + ba_r[...]
        v = jnp.dot(y, wv_r[...], preferred_element_type=jnp.float32)
        ua = lax.bitcast_convert_type(ka, jnp.uint32) + jnp.uint32(0x8000)
        uv = lax.bitcast_convert_type(v, jnp.uint32) + jnp.uint32(0x8000)
        packed = (ua & jnp.uint32(0xFFFF0000)) | (uv >> jnp.uint32(16))
        kv_o[...] = lax.bitcast_convert_type(packed, jnp.int32)

    return pl.pallas_call(
        body,
        grid=(GRID,),
        in_specs=[
            pl.BlockSpec((B, C), lambda i: (i, 0)),
            _full((1, C)), _full((1, C)),
            _full((C, C)), _full((1, C)),
            _full((C, C)), _full((C, C)), _full((1, C)), _full((C, C)),
        ],
        out_specs=[
            pl.BlockSpec((B, C), lambda i: (i, 0)),
            pl.BlockSpec((B, C), lambda i: (i, 0)),
            pl.BlockSpec((B, C), lambda i: (i, 0)),
        ],
        out_shape=[
            jax.ShapeDtypeStruct((N, C), jnp.float32),
            jax.ShapeDtypeStruct((N, C), jnp.float32),
            jax.ShapeDtypeStruct((N, C), jnp.int32),
        ],
        compiler_params=pltpu.CompilerParams(
            dimension_semantics=("arbitrary",)),
    )(x, sc1, sh1, Wl1, bl1, WqA, WkA, ba1, Wv)


def _tc3(s0, kvn, rel16, qA, xn, Wp1p, bp1p, Wp2p, bp2, Wp2Ap, bp2A, Wa2,
         ba2, Wl2, bl2, Ws1, bs1, Ws2, bs2):
    def body(kvn_r, rel_r, qa_r, xn_r, wp1_r, bp1_r, wp2_r, bp2_r, wp2a_r,
             bp2a_r, wa2_r, ba2_r, wl2_r, bl2_r, ws1_r, bs1_r, ws2_r, bs2_r,
             s_o, st_o):
        i = pl.program_id(0)
        rel = rel_r[...]
        e = jax.nn.relu(jnp.dot(rel, wp1_r[...], preferred_element_type=jnp.float32) + bp1_r[...])
        pos = jnp.dot(e, wp2_r[...], preferred_element_type=jnp.float32) + bp2_r[...]
        posA = jnp.dot(e, wp2a_r[...], preferred_element_type=jnp.float32) + bp2a_r[...]
        u = lax.bitcast_convert_type(kvn_r[...], jnp.uint32)
        kan = lax.bitcast_convert_type(u & jnp.uint32(0xFFFF0000), jnp.float32)
        vn = lax.bitcast_convert_type(u << jnp.uint32(16), jnp.float32)
        qrep = jnp.broadcast_to(qa_r[...][:, None, :], (B, K, C)).reshape(BK, C)
        w1 = jax.nn.relu(kan - qrep + posA)
        w = jnp.dot(w1, wa2_r[...], preferred_element_type=jnp.float32) + ba2_r[...]
        w3 = w.reshape(B, K, C)
        m = jnp.max(w3, axis=1, keepdims=True)
        ew = jnp.exp(w3 - m)
        ssum = jnp.sum(ew, axis=1)
        z = (vn + pos).reshape(B, K, C)
        agg = jnp.sum(ew * z, axis=1) / ssum
        x2 = jax.nn.relu(
            xn_r[...] + jnp.dot(agg, wl2_r[...], preferred_element_type=jnp.float32) + bl2_r[...])
        h = jax.nn.gelu(jnp.dot(x2, ws1_r[...], preferred_element_type=jnp.float32) + bs1_r[...])
        s = jnp.dot(h, ws2_r[...], preferred_element_type=jnp.float32) + bs2_r[...]
        s_o[...] = s

        @pl.when(i == 0)
        def _():
            st_o[...] = jnp.zeros((8, C), jnp.float32)

        upd = jnp.concatenate(
            [jnp.sum(s, axis=0, keepdims=True),
             jnp.sum(s * s, axis=0, keepdims=True),
             jnp.zeros((6, C), jnp.float32)], axis=0)
        st_o[...] += upd

    return pl.pallas_call(
        body,
        grid=(SGRID,),
        in_specs=[
            pl.BlockSpec((BK, C), lambda i: (i, 0)),
            pl.BlockSpec((BK, 16), lambda i: (i, 0)),
            pl.BlockSpec((B, C), lambda i: (s0 * SGRID + i, 0)),
            pl.BlockSpec((B, C), lambda i: (s0 * SGRID + i, 0)),
            _full((16, 16)), _full((1, 16)),
            _full((16, C)), _full((1, C)),
            _full((16, C)), _full((1, C)),
            _full((C, C)), _full((1, C)),
            _full((C, C)), _full((1, C)),
            _full((C, C)), _full((1, C)),
            _full((C, C)), _full((1, C)),
        ],
        out_specs=[
            pl.BlockSpec((B, C), lambda i: (i, 0)),
            _full((8, C)),
        ],
        out_shape=[
            jax.ShapeDtypeStruct((SN, C), jnp.float32),
            jax.ShapeDtypeStruct((8, C), jnp.float32),
        ],
        compiler_params=pltpu.CompilerParams(
            dimension_semantics=("arbitrary",)),
    )(kvn, rel16, qA, xn, Wp1p, bp1p, Wp2p, bp2, Wp2Ap, bp2A, Wa2, ba2,
      Wl2, bl2, Ws1, bs1, Ws2, bs2)


def _tc4(s, sc2, sh2):
    B4 = 1000

    def body(s_r, sc_r, sh_r, o_r):
        o_r[...] = jax.nn.gelu(s_r[...] * sc_r[...] + sh_r[...])

    return pl.pallas_call(
        body,
        grid=(N // B4,),
        in_specs=[
            pl.BlockSpec((B4, C), lambda i: (i, 0)),
            _full((1, C)), _full((1, C)),
        ],
        out_specs=pl.BlockSpec((B4, C), lambda i: (i, 0)),
        out_shape=jax.ShapeDtypeStruct((N, C), jnp.float32),
    )(s, sc2, sh2)


def _bn_scale_shift(ssum, ssq, g, b):
    m = ssum / N
    v = ssq / N - m * m
    sc = g / jnp.sqrt(v + 1e-5)
    return sc[None, :], (b - m * sc)[None, :]


def kernel(coord, feat, offset, neighbor_idx, params):
    p = params
    dirs16 = jnp.zeros((16, NG), jnp.float32).at[:3].set(p['gib_dirs'].T)
    sig = jax.nn.softplus(p['gib_sigma']) + 1e-4
    sig2inv = (1.0 / (2.0 * sig * sig))[None, :]
    cvx_smT = jax.nn.softmax(p['cvx'], axis=1).T
    coord16 = jnp.zeros((N, 16), jnp.float32).at[:, :3].set(coord)
    WqA = p['Wq'] @ p['Wa1']
    WkA = p['Wk'] @ p['Wa1']
    Wp1p = jnp.zeros((16, 16), jnp.float32).at[:3, :3].set(p['Wp1'])
    bp1p = jnp.zeros((1, 16), jnp.float32).at[0, :3].set(p['bp1'])
    Wp2p = jnp.zeros((16, C), jnp.float32).at[:3].set(p['Wp2'])
    Wp2Ap = Wp2p @ p['Wa1']
    bp2A = (p['bp2'] @ p['Wa1'])[None, :]

    nbr_flat = neighbor_idx.reshape(-1)

    rel16_s, x_s, st1_s = [], [], []
    for s in range(S):
        cn = _sc_gather(coord16, lax.slice(nbr_flat, (s * SE,), ((s + 1) * SE,)), 16)
        rel16, x, st1 = _tc1(
            s, coord16, cn, feat, dirs16, sig2inv, cvx_smT,
            p['W_enc'], p['b_enc'][None, :], p['W1'], p['b1'][None, :],
            p['W2'], p['b2'][None, :])
        rel16_s.append(rel16)
        x_s.append(x)
        st1_s.append(st1)
    st1 = functools.reduce(lambda a, b: a + b, st1_s)
    sc1, sh1 = _bn_scale_shift(st1[0], st1[1], p['g1'], p['be1'])
    x = jnp.concatenate(x_s, axis=0)

    xn, qA, kv = _tc2(
        x, sc1, sh1, p['Wl1'], p['bl1'][None, :], WqA, WkA,
        p['ba1'][None, :], p['Wv'])

    s_s, st2_s = [], []
    for s in range(S):
        kvn = _sc_gather(kv, lax.slice(nbr_flat, (s * SE,), ((s + 1) * SE,)),
                         C, jnp.int32)
        ss, st2 = _tc3(
            s, kvn, rel16_s[s], qA, xn, Wp1p, bp1p, Wp2p, p['bp2'][None, :],
            Wp2Ap, bp2A, p['Wa2'], p['ba2'][None, :], p['Wl2'],
            p['bl2'][None, :], p['Ws1'], p['bs1'][None, :], p['Ws2'],
            p['bs2'][None, :])
        s_s.append(ss)
        st2_s.append(st2)
    st2 = functools.reduce(lambda a, b: a + b, st2_s)
    sc2, sh2 = _bn_scale_shift(st2[0], st2[1], p['g2'], p['be2'])
    s = jnp.concatenate(s_s, axis=0)

    out = _tc4(s, sc2, sh2)
    return (coord, out, offset)


# speedup vs baseline: 1.3008x; 1.0940x over previous
import functools

import jax
import jax.numpy as jnp
from jax import lax
from jax.experimental import pallas as pl
from jax.experimental.pallas import tpu as pltpu
from jax.experimental.pallas import tpu_sc as plsc

N = 10000
K = 16
C = 128
NG = 32
NO = 64
FE = 16
KR = 0.2
B = 400
BK = B * K
GRID = N // B
S = 1
SN = N // S
SE = SN * K
SGRID = SN // B

_NC = 2
_NS = 16
_NW = _NC * _NS
_CHUNK = 128


def _sc_gather(table, idx, d, dtype=jnp.float32):
    n_idx = idx.shape[0]
    n_chunks = n_idx // _CHUNK
    nfull = n_chunks // _NW
    extra = n_chunks - nfull * _NW
    mesh = plsc.VectorSubcoreMesh(core_axis_name="c", subcore_axis_name="s")

    @functools.partial(
        pl.kernel,
        mesh=mesh,
        out_type=jax.ShapeDtypeStruct((n_idx, d), dtype),
        scratch_types=[
            pltpu.VMEM(((nfull + 1) * _CHUNK,), jnp.int32),
            pltpu.VMEM((_CHUNK, d), dtype),
            pltpu.VMEM((_CHUNK, d), dtype),
            pltpu.SemaphoreType.DMA,
            pltpu.SemaphoreType.DMA,
        ],
        compiler_params=pltpu.CompilerParams(use_tc_tiling_on_sc=(d % 128 == 0)),
    )
    def gather_kernel(table_hbm, idx_hbm, out_hbm, idx_v, buf0, buf1, g0, g1):
        wid = lax.axis_index("s") * _NC + lax.axis_index("c")
        start = wid * nfull + jnp.minimum(wid, extra)
        base = start * _CHUNK

        pltpu.sync_copy(idx_hbm.at[pl.ds(base, nfull * _CHUNK)],
                        idx_v.at[pl.ds(0, nfull * _CHUNK)])

        @pl.when(wid < extra)
        def _():
            pltpu.sync_copy(
                idx_hbm.at[pl.ds(base + nfull * _CHUNK, _CHUNK)],
                idx_v.at[pl.ds(nfull * _CHUNK, _CHUNK)])

        def start_gather(j, buf, sem):
            return pltpu.async_copy(
                table_hbm.at[idx_v.at[pl.ds(j * _CHUNK, _CHUNK)]], buf, sem)

        def writeback(j, buf):
            pltpu.sync_copy(buf, out_hbm.at[pl.ds(base + j * _CHUNK, _CHUNK)])

        start_gather(0, buf0, g0)

        def body(k, carry):
            ja = 2 * k + 1
            jb = 2 * k + 2
            start_gather(ja, buf1, g1)
            pltpu.make_async_copy(table_hbm, buf0, g0).wait()
            writeback(ja - 1, buf0)
            start_gather(jb, buf0, g0)
            pltpu.make_async_copy(table_hbm, buf1, g1).wait()
            writeback(ja, buf1)
            return carry

        lax.fori_loop(0, (nfull - 1) // 2, body, 0)

        @pl.when(wid < extra)
        def _():
            start_gather(nfull, buf1, g1)

        pltpu.make_async_copy(table_hbm, buf0, g0).wait()
        writeback(nfull - 1, buf0)

        @pl.when(wid < extra)
        def _():
            pltpu.make_async_copy(table_hbm, buf1, g1).wait()
            writeback(nfull, buf1)

    return gather_kernel(table, idx)


def _full(shape):
    return pl.BlockSpec(shape, lambda i: (0, 0))


def _tc1(s0, coord16, cn, feat, dirs16, sig2inv, cvx_smT, W_enc, b_enc, W1,
         b1, W2, b2):
    kfac = 1.0 / (2.0 * KR * KR)

    def body(coord_r, cn_r, feat_r, dirs_r, s2i_r, cvx_r, we_r, be_r, w1_r,
             bb1_r, w2_r, bb2_r, rel_o, x_o, st_o):
        i = pl.program_id(0)
        rel3 = cn_r[...] - coord_r[...][None, :, :]
        rel_o[...] = rel3
        rel = rel3.reshape(K * B, 16)
        d2 = jnp.sum(rel * rel, axis=1, keepdims=True)
        proj = jnp.dot(rel, dirs_r[...], preferred_element_type=jnp.float32)
        resp = jnp.exp(-(proj * proj) * s2i_r[...]) * jnp.exp(-d2 * kfac)
        gib = jnp.mean(resp.reshape(K, B, NG), axis=0)
        obs = jnp.dot(gib, cvx_r[...], preferred_element_type=jnp.float32)
        feat = feat_r[...]
        fenc = jnp.dot(feat, we_r[...], preferred_element_type=jnp.float32) + be_r[...]
        g = jnp.concatenate([fenc, obs], axis=1)
        h = jax.nn.gelu(jnp.dot(g, w1_r[...], preferred_element_type=jnp.float32) + bb1_r[...])
        g2 = jnp.dot(h, w2_r[...], preferred_element_type=jnp.float32) + bb2_r[...]
        x = feat + g2
        x_o[...] = x

        @pl.when(i == 0)
        def _():
            st_o[...] = jnp.zeros((8, C), jnp.float32)

        upd = jnp.concatenate(
            [jnp.sum(x, axis=0, keepdims=True),
             jnp.sum(x * x, axis=0, keepdims=True),
             jnp.zeros((6, C), jnp.float32)], axis=0)
        st_o[...] += upd

    return pl.pallas_call(
        body,
        grid=(SGRID,),
        in_specs=[
            pl.BlockSpec((B, 16), lambda i: (s0 * SGRID + i, 0)),
            pl.BlockSpec((K, B, 16), lambda i: (0, i, 0)),
            pl.BlockSpec((B, C), lambda i: (s0 * SGRID + i, 0)),
            _full((16, NG)), _full((1, NG)), _full((NG, NO)),
            _full((C, FE)), _full((1, FE)),
            _full((FE + NO, FE + NO)), _full((1, FE + NO)),
            _full((FE + NO, C)), _full((1, C)),
        ],
        out_specs=[
            pl.BlockSpec((K, B, 16), lambda i: (0, i, 0)),
            pl.BlockSpec((B, C), lambda i: (i, 0)),
            _full((8, C)),
        ],
        out_shape=[
            jax.ShapeDtypeStruct((K, SN, 16), jnp.float32),
            jax.ShapeDtypeStruct((SN, C), jnp.float32),
            jax.ShapeDtypeStruct((8, C), jnp.float32),
        ],
        compiler_params=pltpu.CompilerParams(
            dimension_semantics=("arbitrary",)),
    )(coord16, cn, feat, dirs16, sig2inv, cvx_smT, W_enc, b_enc, W1, b1,
      W2, b2)


def _tc2(x, sc1, sh1, Wl1, bl1, WqA, WkA, ba1, Wv):
    def body(x_r, sc_r, sh_r, wl_r, bl_r, wq_r, wk_r, ba_r, wv_r,
             xn_o, qa_o, kv_o):
        xn = jax.nn.gelu(x_r[...] * sc_r[...] + sh_r[...])
        xn_o[...] = xn
        y = jnp.dot(xn, wl_r[...], preferred_element_type=jnp.float32) + bl_r[...]
        qa_o[...] = jnp.dot(y, wq_r[...], preferred_element_type=jnp.float32)
        ka = jnp.dot(y, wk_r[...], preferred_element_type=jnp.float32) + ba_r[...]
        v = jnp.dot(y, wv_r[...], preferred_element_type=jnp.float32)
        ua = lax.bitcast_convert_type(ka, jnp.uint32) + jnp.uint32(0x8000)
        uv = lax.bitcast_convert_type(v, jnp.uint32) + jnp.uint32(0x8000)
        packed = (ua & jnp.uint32(0xFFFF0000)) | (uv >> jnp.uint32(16))
        kv_o[...] = lax.bitcast_convert_type(packed, jnp.int32)

    return pl.pallas_call(
        body,
        grid=(GRID,),
        in_specs=[
            pl.BlockSpec((B, C), lambda i: (i, 0)),
            _full((1, C)), _full((1, C)),
            _full((C, C)), _full((1, C)),
            _full((C, C)), _full((C, C)), _full((1, C)), _full((C, C)),
        ],
        out_specs=[
            pl.BlockSpec((B, C), lambda i: (i, 0)),
            pl.BlockSpec((B, C), lambda i: (i, 0)),
            pl.BlockSpec((B, C), lambda i: (i, 0)),
        ],
        out_shape=[
            jax.ShapeDtypeStruct((N, C), jnp.float32),
            jax.ShapeDtypeStruct((N, C), jnp.float32),
            jax.ShapeDtypeStruct((N, C), jnp.int32),
        ],
        compiler_params=pltpu.CompilerParams(
            dimension_semantics=("arbitrary",)),
    )(x, sc1, sh1, Wl1, bl1, WqA, WkA, ba1, Wv)


def _tc3(s0, kvn, rel16, qA, xn, Wp1p, bp1p, Wp2p, bp2, Wp2Ap, bp2A, Wa2,
         ba2, Wl2, bl2, Ws1, bs1, Ws2, bs2):
    def body(kvn_r, rel_r, qa_r, xn_r, wp1_r, bp1_r, wp2_r, bp2_r, wp2a_r,
             bp2a_r, wa2_r, ba2_r, wl2_r, bl2_r, ws1_r, bs1_r, ws2_r, bs2_r,
             s_o, st_o):
        i = pl.program_id(0)
        rel = rel_r[...].reshape(K * B, 16)
        e = jax.nn.relu(jnp.dot(rel, wp1_r[...], preferred_element_type=jnp.float32) + bp1_r[...])
        pos = (jnp.dot(e, wp2_r[...], preferred_element_type=jnp.float32)
               + bp2_r[...]).reshape(K, B, C)
        posA = (jnp.dot(e, wp2a_r[...], preferred_element_type=jnp.float32)
                + bp2a_r[...]).reshape(K, B, C)
        u = lax.bitcast_convert_type(kvn_r[...], jnp.uint32)
        kan = lax.bitcast_convert_type(u & jnp.uint32(0xFFFF0000), jnp.float32)
        vn = lax.bitcast_convert_type(u << jnp.uint32(16), jnp.float32)
        w1 = jax.nn.relu(kan - qa_r[...][None, :, :] + posA)
        w3 = jnp.dot(w1.reshape(K * B, C), wa2_r[...],
                     preferred_element_type=jnp.float32).reshape(K, B, C) + ba2_r[...]
        m = jnp.max(w3, axis=0, keepdims=True)
        ew = jnp.exp(w3 - m)
        ssum = jnp.sum(ew, axis=0)
        z = vn + pos
        agg = jnp.sum(ew * z, axis=0) / ssum
        x2 = jax.nn.relu(
            xn_r[...] + jnp.dot(agg, wl2_r[...], preferred_element_type=jnp.float32) + bl2_r[...])
        h = jax.nn.gelu(jnp.dot(x2, ws1_r[...], preferred_element_type=jnp.float32) + bs1_r[...])
        s = jnp.dot(h, ws2_r[...], preferred_element_type=jnp.float32) + bs2_r[...]
        s_o[...] = s

        @pl.when(i == 0)
        def _():
            st_o[...] = jnp.zeros((8, C), jnp.float32)

        upd = jnp.concatenate(
            [jnp.sum(s, axis=0, keepdims=True),
             jnp.sum(s * s, axis=0, keepdims=True),
             jnp.zeros((6, C), jnp.float32)], axis=0)
        st_o[...] += upd

    return pl.pallas_call(
        body,
        grid=(SGRID,),
        in_specs=[
            pl.BlockSpec((K, B, C), lambda i: (0, i, 0)),
            pl.BlockSpec((K, B, 16), lambda i: (0, i, 0)),
            pl.BlockSpec((B, C), lambda i: (s0 * SGRID + i, 0)),
            pl.BlockSpec((B, C), lambda i: (s0 * SGRID + i, 0)),
            _full((16, 16)), _full((1, 16)),
            _full((16, C)), _full((1, C)),
            _full((16, C)), _full((1, C)),
            _full((C, C)), _full((1, C)),
            _full((C, C)), _full((1, C)),
            _full((C, C)), _full((1, C)),
            _full((C, C)), _full((1, C)),
        ],
        out_specs=[
            pl.BlockSpec((B, C), lambda i: (i, 0)),
            _full((8, C)),
        ],
        out_shape=[
            jax.ShapeDtypeStruct((SN, C), jnp.float32),
            jax.ShapeDtypeStruct((8, C), jnp.float32),
        ],
        compiler_params=pltpu.CompilerParams(
            dimension_semantics=("arbitrary",)),
    )(kvn, rel16, qA, xn, Wp1p, bp1p, Wp2p, bp2, Wp2Ap, bp2A, Wa2, ba2,
      Wl2, bl2, Ws1, bs1, Ws2, bs2)


def _tc4(s, sc2, sh2):
    B4 = 1000

    def body(s_r, sc_r, sh_r, o_r):
        o_r[...] = jax.nn.gelu(s_r[...] * sc_r[...] + sh_r[...])

    return pl.pallas_call(
        body,
        grid=(N // B4,),
        in_specs=[
            pl.BlockSpec((B4, C), lambda i: (i, 0)),
            _full((1, C)), _full((1, C)),
        ],
        out_specs=pl.BlockSpec((B4, C), lambda i: (i, 0)),
        out_shape=jax.ShapeDtypeStruct((N, C), jnp.float32),
    )(s, sc2, sh2)


def _bn_scale_shift(ssum, ssq, g, b):
    m = ssum / N
    v = ssq / N - m * m
    sc = g / jnp.sqrt(v + 1e-5)
    return sc[None, :], (b - m * sc)[None, :]


def kernel(coord, feat, offset, neighbor_idx, params):
    p = params
    dirs16 = jnp.zeros((16, NG), jnp.float32).at[:3].set(p['gib_dirs'].T)
    sig = jax.nn.softplus(p['gib_sigma']) + 1e-4
    sig2inv = (1.0 / (2.0 * sig * sig))[None, :]
    cvx_smT = jax.nn.softmax(p['cvx'], axis=1).T
    coord16 = jnp.zeros((N, 16), jnp.float32).at[:, :3].set(coord)
    WqA = p['Wq'] @ p['Wa1']
    WkA = p['Wk'] @ p['Wa1']
    Wp1p = jnp.zeros((16, 16), jnp.float32).at[:3, :3].set(p['Wp1'])
    bp1p = jnp.zeros((1, 16), jnp.float32).at[0, :3].set(p['bp1'])
    Wp2p = jnp.zeros((16, C), jnp.float32).at[:3].set(p['Wp2'])
    Wp2Ap = Wp2p @ p['Wa1']
    bp2A = (p['bp2'] @ p['Wa1'])[None, :]

    nbr_flat = neighbor_idx.T.reshape(-1)

    rel16_s, x_s, st1_s = [], [], []
    for s in range(S):
        cn = _sc_gather(coord16, lax.slice(nbr_flat, (s * SE,), ((s + 1) * SE,)),
                        16).reshape(K, SN, 16)
        rel16, x, st1 = _tc1(
            s, coord16, cn, feat, dirs16, sig2inv, cvx_smT,
            p['W_enc'], p['b_enc'][None, :], p['W1'], p['b1'][None, :],
            p['W2'], p['b2'][None, :])
        rel16_s.append(rel16)
        x_s.append(x)
        st1_s.append(st1)
    st1 = functools.reduce(lambda a, b: a + b, st1_s)
    sc1, sh1 = _bn_scale_shift(st1[0], st1[1], p['g1'], p['be1'])
    x = jnp.concatenate(x_s, axis=0)

    xn, qA, kv = _tc2(
        x, sc1, sh1, p['Wl1'], p['bl1'][None, :], WqA, WkA,
        p['ba1'][None, :], p['Wv'])

    s_s, st2_s = [], []
    for s in range(S):
        kvn = _sc_gather(kv, lax.slice(nbr_flat, (s * SE,), ((s + 1) * SE,)),
                         C, jnp.int32).reshape(K, SN, C)
        ss, st2 = _tc3(
            s, kvn, rel16_s[s], qA, xn, Wp1p, bp1p, Wp2p, p['bp2'][None, :],
            Wp2Ap, bp2A, p['Wa2'], p['ba2'][None, :], p['Wl2'],
            p['bl2'][None, :], p['Ws1'], p['bs1'][None, :], p['Ws2'],
            p['bs2'][None, :])
        s_s.append(ss)
        st2_s.append(st2)
    st2 = functools.reduce(lambda a, b: a + b, st2_s)
    sc2, sh2 = _bn_scale_shift(st2[0], st2[1], p['g2'], p['be2'])
    s = jnp.concatenate(s_s, axis=0)

    out = _tc4(s, sc2, sh2)
    return (coord, out, offset)


# speedup vs baseline: 1.3471x; 1.0356x over previous
import functools

import jax
import jax.numpy as jnp
from jax import lax
from jax.experimental import pallas as pl
from jax.experimental.pallas import tpu as pltpu
from jax.experimental.pallas import tpu_sc as plsc

N = 10000
K = 16
C = 128
NG = 32
NO = 64
FE = 16
KR = 0.2
B = 400
BK = B * K
GRID = N // B
S = 1
SN = N // S
SE = SN * K
SGRID = SN // B

_NC = 2
_NS = 16
_NW = _NC * _NS
_CHUNK = 128


def _sc_gather(table, idx, d, dtype=jnp.float32):
    n_idx = idx.shape[0]
    n_chunks = n_idx // _CHUNK
    nfull = n_chunks // _NW
    extra = n_chunks - nfull * _NW
    mesh = plsc.VectorSubcoreMesh(core_axis_name="c", subcore_axis_name="s")

    @functools.partial(
        pl.kernel,
        mesh=mesh,
        out_type=jax.ShapeDtypeStruct((n_idx, d), dtype),
        scratch_types=[
            pltpu.VMEM(((nfull + 1) * _CHUNK,), jnp.int32),
            pltpu.VMEM((_CHUNK, d), dtype),
            pltpu.VMEM((_CHUNK, d), dtype),
            pltpu.SemaphoreType.DMA,
            pltpu.SemaphoreType.DMA,
        ],
        compiler_params=pltpu.CompilerParams(use_tc_tiling_on_sc=(d % 128 == 0)),
    )
    def gather_kernel(table_hbm, idx_hbm, out_hbm, idx_v, buf0, buf1, g0, g1):
        wid = lax.axis_index("s") * _NC + lax.axis_index("c")
        start = wid * nfull + jnp.minimum(wid, extra)
        base = start * _CHUNK

        pltpu.sync_copy(idx_hbm.at[pl.ds(base, nfull * _CHUNK)],
                        idx_v.at[pl.ds(0, nfull * _CHUNK)])

        @pl.when(wid < extra)
        def _():
            pltpu.sync_copy(
                idx_hbm.at[pl.ds(base + nfull * _CHUNK, _CHUNK)],
                idx_v.at[pl.ds(nfull * _CHUNK, _CHUNK)])

        def start_gather(j, buf, sem):
            return pltpu.async_copy(
                table_hbm.at[idx_v.at[pl.ds(j * _CHUNK, _CHUNK)]], buf, sem)

        def writeback(j, buf):
            pltpu.sync_copy(buf, out_hbm.at[pl.ds(base + j * _CHUNK, _CHUNK)])

        start_gather(0, buf0, g0)

        def body(k, carry):
            ja = 2 * k + 1
            jb = 2 * k + 2
            start_gather(ja, buf1, g1)
            pltpu.make_async_copy(table_hbm, buf0, g0).wait()
            writeback(ja - 1, buf0)
            start_gather(jb, buf0, g0)
            pltpu.make_async_copy(table_hbm, buf1, g1).wait()
            writeback(ja, buf1)
            return carry

        lax.fori_loop(0, (nfull - 1) // 2, body, 0)

        @pl.when(wid < extra)
        def _():
            start_gather(nfull, buf1, g1)

        pltpu.make_async_copy(table_hbm, buf0, g0).wait()
        writeback(nfull - 1, buf0)

        @pl.when(wid < extra)
        def _():
            pltpu.make_async_copy(table_hbm, buf1, g1).wait()
            writeback(nfull, buf1)

    return gather_kernel(table, idx)


def _full(shape):
    return pl.BlockSpec(shape, lambda i: (0, 0))


def _tc1(s0, coord16, cn, feat, dirs16, sig2inv, cvx_smT, W_enc, b_enc, W1,
         b1, W2, b2):
    kfac = 1.0 / (2.0 * KR * KR)

    def body(coord_r, cn_r, feat_r, dirs_r, s2i_r, cvx_r, we_r, be_r, w1_r,
             bb1_r, w2_r, bb2_r, rel_o, x_o, st_o):
        i = pl.program_id(0)
        rel3 = cn_r[...] - coord_r[...][None, :, :]
        rel_o[...] = rel3.astype(jnp.bfloat16)
        rel = rel3.reshape(K * B, 16)
        d2 = jnp.sum(rel * rel, axis=1, keepdims=True)
        proj = jnp.dot(rel, dirs_r[...], preferred_element_type=jnp.float32)
        resp = jnp.exp(-(proj * proj) * s2i_r[...] - d2 * kfac)
        gib = jnp.mean(resp.reshape(K, B, NG), axis=0)
        obs = jnp.dot(gib, cvx_r[...], preferred_element_type=jnp.float32)
        feat = feat_r[...]
        fenc = jnp.dot(feat, we_r[...], preferred_element_type=jnp.float32) + be_r[...]
        g = jnp.concatenate([fenc, obs], axis=1)
        h = jax.nn.gelu(jnp.dot(g, w1_r[...], preferred_element_type=jnp.float32) + bb1_r[...])
        g2 = jnp.dot(h, w2_r[...], preferred_element_type=jnp.float32) + bb2_r[...]
        x = feat + g2
        x_o[...] = x

        @pl.when(i == 0)
        def _():
            st_o[...] = jnp.zeros((8, C), jnp.float32)

        upd = jnp.concatenate(
            [jnp.sum(x, axis=0, keepdims=True),
             jnp.sum(x * x, axis=0, keepdims=True),
             jnp.zeros((6, C), jnp.float32)], axis=0)
        st_o[...] += upd

    return pl.pallas_call(
        body,
        grid=(SGRID,),
        in_specs=[
            pl.BlockSpec((B, 16), lambda i: (s0 * SGRID + i, 0)),
            pl.BlockSpec((K, B, 16), lambda i: (0, i, 0)),
            pl.BlockSpec((B, C), lambda i: (s0 * SGRID + i, 0)),
            _full((16, NG)), _full((1, NG)), _full((NG, NO)),
            _full((C, FE)), _full((1, FE)),
            _full((FE + NO, FE + NO)), _full((1, FE + NO)),
            _full((FE + NO, C)), _full((1, C)),
        ],
        out_specs=[
            pl.BlockSpec((K, B, 16), lambda i: (0, i, 0)),
            pl.BlockSpec((B, C), lambda i: (i, 0)),
            _full((8, C)),
        ],
        out_shape=[
            jax.ShapeDtypeStruct((K, SN, 16), jnp.bfloat16),
            jax.ShapeDtypeStruct((SN, C), jnp.float32),
            jax.ShapeDtypeStruct((8, C), jnp.float32),
        ],
        compiler_params=pltpu.CompilerParams(
            dimension_semantics=("arbitrary",)),
    )(coord16, cn, feat, dirs16, sig2inv, cvx_smT, W_enc, b_enc, W1, b1,
      W2, b2)


def _tc2(x, sc1, sh1, Wl1, bl1, WqA, WkA, ba1, Wv, bp2A, bp2):
    def body(x_r, sc_r, sh_r, wl_r, bl_r, wq_r, wk_r, ba_r, wv_r, bp2a_r,
             bp2_r, xn_o, qa_o, kv_o):
        xn = jax.nn.gelu(x_r[...] * sc_r[...] + sh_r[...])
        xn_o[...] = xn
        y = jnp.dot(xn, wl_r[...], preferred_element_type=jnp.float32) + bl_r[...]
        qa_o[...] = jnp.dot(y, wq_r[...], preferred_element_type=jnp.float32) - bp2a_r[...]
        ka = jnp.dot(y, wk_r[...], preferred_element_type=jnp.float32) + ba_r[...]
        v = jnp.dot(y, wv_r[...], preferred_element_type=jnp.float32) + bp2_r[...]
        ua = lax.bitcast_convert_type(ka, jnp.uint32) + jnp.uint32(0x8000)
        uv = lax.bitcast_convert_type(v, jnp.uint32) + jnp.uint32(0x8000)
        packed = (ua & jnp.uint32(0xFFFF0000)) | (uv >> jnp.uint32(16))
        kv_o[...] = lax.bitcast_convert_type(packed, jnp.int32)

    return pl.pallas_call(
        body,
        grid=(GRID,),
        in_specs=[
            pl.BlockSpec((B, C), lambda i: (i, 0)),
            _full((1, C)), _full((1, C)),
            _full((C, C)), _full((1, C)),
            _full((C, C)), _full((C, C)), _full((1, C)), _full((C, C)),
            _full((1, C)), _full((1, C)),
        ],
        out_specs=[
            pl.BlockSpec((B, C), lambda i: (i, 0)),
            pl.BlockSpec((B, C), lambda i: (i, 0)),
            pl.BlockSpec((B, C), lambda i: (i, 0)),
        ],
        out_shape=[
            jax.ShapeDtypeStruct((N, C), jnp.float32),
            jax.ShapeDtypeStruct((N, C), jnp.float32),
            jax.ShapeDtypeStruct((N, C), jnp.int32),
        ],
        compiler_params=pltpu.CompilerParams(
            dimension_semantics=("arbitrary",)),
    )(x, sc1, sh1, Wl1, bl1, WqA, WkA, ba1, Wv, bp2A, bp2)


def _tc3(s0, kvn, rel16, qA, xn, Wp1p, bp1p, Wp2p, Wp2Ap, Wa2,
         Wl2, bl2, Ws1, bs1, Ws2, bs2):
    def body(kvn_r, rel_r, qa_r, xn_r, wp1_r, bp1_r, wp2_r, wp2a_r,
             wa2_r, wl2_r, bl2_r, ws1_r, bs1_r, ws2_r, bs2_r,
             s_o, st_o):
        i = pl.program_id(0)
        rel = rel_r[...].reshape(K * B, 16)
        e = jax.nn.relu(jnp.dot(rel, wp1_r[...], preferred_element_type=jnp.float32)
                        + bp1_r[...]).astype(jnp.bfloat16)
        pos = jnp.dot(e, wp2_r[...],
                      preferred_element_type=jnp.float32).reshape(K, B, C)
        posA = jnp.dot(e, wp2a_r[...],
                       preferred_element_type=jnp.float32).reshape(K, B, C)
        u = lax.bitcast_convert_type(kvn_r[...], jnp.uint32)
        kan = lax.bitcast_convert_type(u & jnp.uint32(0xFFFF0000), jnp.float32)
        vn = lax.bitcast_convert_type(u << jnp.uint32(16), jnp.float32)
        w1 = jax.nn.relu(kan - qa_r[...][None, :, :] + posA).astype(jnp.bfloat16)
        w3 = jnp.dot(w1.reshape(K * B, C), wa2_r[...],
                     preferred_element_type=jnp.float32).reshape(K, B, C)
        m = jnp.max(w3, axis=0, keepdims=True)
        ew = jnp.exp(w3 - m)
        ssum = jnp.sum(ew, axis=0)
        z = vn + pos
        agg = jnp.sum(ew * z, axis=0) / ssum
        x2 = jax.nn.relu(
            xn_r[...] + jnp.dot(agg, wl2_r[...], preferred_element_type=jnp.float32) + bl2_r[...])
        h = jax.nn.gelu(jnp.dot(x2, ws1_r[...], preferred_element_type=jnp.float32) + bs1_r[...])
        s = jnp.dot(h, ws2_r[...], preferred_element_type=jnp.float32) + bs2_r[...]
        s_o[...] = s

        @pl.when(i == 0)
        def _():
            st_o[...] = jnp.zeros((8, C), jnp.float32)

        upd = jnp.concatenate(
            [jnp.sum(s, axis=0, keepdims=True),
             jnp.sum(s * s, axis=0, keepdims=True),
             jnp.zeros((6, C), jnp.float32)], axis=0)
        st_o[...] += upd

    return pl.pallas_call(
        body,
        grid=(SGRID,),
        in_specs=[
            pl.BlockSpec((K, B, C), lambda i: (0, i, 0)),
            pl.BlockSpec((K, B, 16), lambda i: (0, i, 0)),
            pl.BlockSpec((B, C), lambda i: (s0 * SGRID + i, 0)),
            pl.BlockSpec((B, C), lambda i: (s0 * SGRID + i, 0)),
            _full((16, 16)), _full((1, 16)),
            _full((16, C)),
            _full((16, C)),
            _full((C, C)),
            _full((C, C)), _full((1, C)),
            _full((C, C)), _full((1, C)),
            _full((C, C)), _full((1, C)),
        ],
        out_specs=[
            pl.BlockSpec((B, C), lambda i: (i, 0)),
            _full((8, C)),
        ],
        out_shape=[
            jax.ShapeDtypeStruct((SN, C), jnp.float32),
            jax.ShapeDtypeStruct((8, C), jnp.float32),
        ],
        compiler_params=pltpu.CompilerParams(
            dimension_semantics=("arbitrary",)),
    )(kvn, rel16, qA, xn, Wp1p, bp1p, Wp2p, Wp2Ap, Wa2,
      Wl2, bl2, Ws1, bs1, Ws2, bs2)


def _tc4(s, sc2, sh2):
    B4 = 1000

    def body(s_r, sc_r, sh_r, o_r):
        o_r[...] = jax.nn.gelu(s_r[...] * sc_r[...] + sh_r[...])

    return pl.pallas_call(
        body,
        grid=(N // B4,),
        in_specs=[
            pl.BlockSpec((B4, C), lambda i: (i, 0)),
            _full((1, C)), _full((1, C)),
        ],
        out_specs=pl.BlockSpec((B4, C), lambda i: (i, 0)),
        out_shape=jax.ShapeDtypeStruct((N, C), jnp.float32),
    )(s, sc2, sh2)


def _bn_scale_shift(ssum, ssq, g, b):
    m = ssum / N
    v = ssq / N - m * m
    sc = g / jnp.sqrt(v + 1e-5)
    return sc[None, :], (b - m * sc)[None, :]


def kernel(coord, feat, offset, neighbor_idx, params):
    p = params
    dirs16 = jnp.zeros((16, NG), jnp.float32).at[:3].set(p['gib_dirs'].T)
    sig = jax.nn.softplus(p['gib_sigma']) + 1e-4
    sig2inv = (1.0 / (2.0 * sig * sig))[None, :]
    cvx_smT = jax.nn.softmax(p['cvx'], axis=1).T
    coord16 = jnp.zeros((N, 16), jnp.float32).at[:, :3].set(coord)
    WqA = p['Wq'] @ p['Wa1']
    WkA = p['Wk'] @ p['Wa1']
    Wp1p = jnp.zeros((16, 16), jnp.float32).at[:3, :3].set(p['Wp1']).astype(jnp.bfloat16)
    bp1p = jnp.zeros((1, 16), jnp.float32).at[0, :3].set(p['bp1'])
    Wp2p = jnp.zeros((16, C), jnp.float32).at[:3].set(p['Wp2'])
    Wp2Ap = (Wp2p @ p['Wa1']).astype(jnp.bfloat16)
    bp2A = (p['bp2'] @ p['Wa1'])[None, :]
    Wp2p = Wp2p.astype(jnp.bfloat16)

    nbr_flat = neighbor_idx.T.reshape(-1)

    rel16_s, x_s, st1_s = [], [], []
    for s in range(S):
        cn = _sc_gather(coord16, lax.slice(nbr_flat, (s * SE,), ((s + 1) * SE,)),
                        16).reshape(K, SN, 16)
        rel16, x, st1 = _tc1(
            s, coord16, cn, feat, dirs16, sig2inv, cvx_smT,
            p['W_enc'], p['b_enc'][None, :], p['W1'], p['b1'][None, :],
            p['W2'], p['b2'][None, :])
        rel16_s.append(rel16)
        x_s.append(x)
        st1_s.append(st1)
    st1 = functools.reduce(lambda a, b: a + b, st1_s)
    sc1, sh1 = _bn_scale_shift(st1[0], st1[1], p['g1'], p['be1'])
    x = jnp.concatenate(x_s, axis=0)

    xn, qA, kv = _tc2(
        x, sc1, sh1, p['Wl1'], p['bl1'][None, :], WqA, WkA,
        p['ba1'][None, :], p['Wv'], bp2A, p['bp2'][None, :])

    s_s, st2_s = [], []
    for s in range(S):
        kvn = _sc_gather(kv, lax.slice(nbr_flat, (s * SE,), ((s + 1) * SE,)),
                         C, jnp.int32).reshape(K, SN, C)
        ss, st2 = _tc3(
            s, kvn, rel16_s[s], qA, xn, Wp1p, bp1p, Wp2p,
            Wp2Ap, p['Wa2'].astype(jnp.bfloat16),
            p['Wl2'],
            p['bl2'][None, :], p['Ws1'], p['bs1'][None, :], p['Ws2'],
            p['bs2'][None, :])
        s_s.append(ss)
        st2_s.append(st2)
    st2 = functools.reduce(lambda a, b: a + b, st2_s)
    sc2, sh2 = _bn_scale_shift(st2[0], st2[1], p['g2'], p['be2'])
    s = jnp.concatenate(s_s, axis=0)

    out = _tc4(s, sc2, sh2)
    return (coord, out, offset)


# speedup vs baseline: 1.3930x; 1.0340x over previous
import functools

import jax
import jax.numpy as jnp
from jax import lax
from jax.experimental import pallas as pl
from jax.experimental.pallas import tpu as pltpu
from jax.experimental.pallas import tpu_sc as plsc

N = 10000
K = 16
C = 128
NG = 32
NO = 64
FE = 16
KR = 0.2
B = 400
BK = B * K
GRID = N // B
S = 1
SN = N // S
SE = SN * K
SGRID = SN // B

_NC = 2
_NS = 16
_NW = _NC * _NS
_CHUNK = 128


def _sc_gather(table, idx, d, dtype=jnp.float32):
    n_idx = idx.shape[0]
    n_chunks = n_idx // _CHUNK
    nfull = n_chunks // _NW
    extra = n_chunks - nfull * _NW
    mesh = plsc.VectorSubcoreMesh(core_axis_name="c", subcore_axis_name="s")

    @functools.partial(
        pl.kernel,
        mesh=mesh,
        out_type=jax.ShapeDtypeStruct((n_idx, d), dtype),
        scratch_types=[
            pltpu.VMEM(((nfull + 1) * _CHUNK,), jnp.int32),
            pltpu.VMEM((_CHUNK, d), dtype),
            pltpu.VMEM((_CHUNK, d), dtype),
            pltpu.SemaphoreType.DMA,
            pltpu.SemaphoreType.DMA,
        ],
        compiler_params=pltpu.CompilerParams(use_tc_tiling_on_sc=(d % 128 == 0)),
    )
    def gather_kernel(table_hbm, idx_hbm, out_hbm, idx_v, buf0, buf1, g0, g1):
        wid = lax.axis_index("s") * _NC + lax.axis_index("c")
        start = wid * nfull + jnp.minimum(wid, extra)
        base = start * _CHUNK

        pltpu.sync_copy(idx_hbm.at[pl.ds(base, nfull * _CHUNK)],
                        idx_v.at[pl.ds(0, nfull * _CHUNK)])

        @pl.when(wid < extra)
        def _():
            pltpu.sync_copy(
                idx_hbm.at[pl.ds(base + nfull * _CHUNK, _CHUNK)],
                idx_v.at[pl.ds(nfull * _CHUNK, _CHUNK)])

        def start_gather(j, buf, sem):
            return pltpu.async_copy(
                table_hbm.at[idx_v.at[pl.ds(j * _CHUNK, _CHUNK)]], buf, sem)

        def writeback(j, buf):
            pltpu.sync_copy(buf, out_hbm.at[pl.ds(base + j * _CHUNK, _CHUNK)])

        start_gather(0, buf0, g0)

        def body(k, carry):
            ja = 2 * k + 1
            jb = 2 * k + 2
            start_gather(ja, buf1, g1)
            pltpu.make_async_copy(table_hbm, buf0, g0).wait()
            writeback(ja - 1, buf0)
            start_gather(jb, buf0, g0)
            pltpu.make_async_copy(table_hbm, buf1, g1).wait()
            writeback(ja, buf1)
            return carry

        lax.fori_loop(0, (nfull - 1) // 2, body, 0)

        @pl.when(wid < extra)
        def _():
            start_gather(nfull, buf1, g1)

        pltpu.make_async_copy(table_hbm, buf0, g0).wait()
        writeback(nfull - 1, buf0)

        @pl.when(wid < extra)
        def _():
            pltpu.make_async_copy(table_hbm, buf1, g1).wait()
            writeback(nfull, buf1)

    return gather_kernel(table, idx)


def _full(shape):
    return pl.BlockSpec(shape, lambda i: (0, 0))


def _tc1(s0, coord16, cn, feat, dirs16, sig2inv, cvx_smT, W_enc, b_enc, W1,
         b1, W2, b2):
    kfac = 1.0 / (2.0 * KR * KR)

    def body(coord_r, cn_r, feat_r, dirs_r, s2i_r, cvx_r, we_r, be_r, w1_r,
             bb1_r, w2_r, bb2_r, rel_o, x_o, st_o):
        i = pl.program_id(0)
        rel3 = cn_r[...] - coord_r[...][None, :, :]
        rel_o[...] = rel3.astype(jnp.bfloat16)
        rel = rel3.reshape(K * B, 16)
        d2 = jnp.sum(rel * rel, axis=1, keepdims=True)
        proj = jnp.dot(rel, dirs_r[...], preferred_element_type=jnp.float32)
        resp = jnp.exp(-(proj * proj) * s2i_r[...] - d2 * kfac)
        gib = jnp.mean(resp.reshape(K, B, NG), axis=0)
        obs = jnp.dot(gib, cvx_r[...], preferred_element_type=jnp.float32)
        feat = feat_r[...]
        fenc = jnp.dot(feat, we_r[...], preferred_element_type=jnp.float32) + be_r[...]
        g = jnp.concatenate([fenc, obs], axis=1)
        h = jax.nn.gelu(jnp.dot(g, w1_r[...], preferred_element_type=jnp.float32) + bb1_r[...])
        g2 = jnp.dot(h, w2_r[...], preferred_element_type=jnp.float32) + bb2_r[...]
        x = feat + g2
        x_o[...] = x

        @pl.when(i == 0)
        def _():
            st_o[...] = jnp.zeros((8, C), jnp.float32)

        upd = jnp.concatenate(
            [jnp.sum(x, axis=0, keepdims=True),
             jnp.sum(x * x, axis=0, keepdims=True),
             jnp.zeros((6, C), jnp.float32)], axis=0)
        st_o[...] += upd

    return pl.pallas_call(
        body,
        grid=(SGRID,),
        in_specs=[
            pl.BlockSpec((B, 16), lambda i: (s0 * SGRID + i, 0)),
            pl.BlockSpec((K, B, 16), lambda i: (0, i, 0)),
            pl.BlockSpec((B, C), lambda i: (s0 * SGRID + i, 0)),
            _full((16, NG)), _full((1, NG)), _full((NG, NO)),
            _full((C, FE)), _full((1, FE)),
            _full((FE + NO, FE + NO)), _full((1, FE + NO)),
            _full((FE + NO, C)), _full((1, C)),
        ],
        out_specs=[
            pl.BlockSpec((K, B, 16), lambda i: (0, i, 0)),
            pl.BlockSpec((B, C), lambda i: (i, 0)),
            _full((8, C)),
        ],
        out_shape=[
            jax.ShapeDtypeStruct((K, SN, 16), jnp.bfloat16),
            jax.ShapeDtypeStruct((SN, C), jnp.float32),
            jax.ShapeDtypeStruct((8, C), jnp.float32),
        ],
        compiler_params=pltpu.CompilerParams(
            dimension_semantics=("arbitrary",)),
    )(coord16, cn, feat, dirs16, sig2inv, cvx_smT, W_enc, b_enc, W1, b1,
      W2, b2)


def _tc2(x, st1, g1, be1, Wl1, bl1, WqA, WkA, ba1, Wv, bp2A, bp2):
    def body(x_r, st_r, g_r, be_r, wl_r, bl_r, wq_r, wk_r, ba_r, wv_r,
             bp2a_r, bp2_r, xn_o, qa_o, kv_o):
        st = st_r[...]
        m = st[0:1] * (1.0 / N)
        var = st[1:2] * (1.0 / N) - m * m
        sc = g_r[...] * lax.rsqrt(var + 1e-5)
        sh = be_r[...] - m * sc
        xn = jax.nn.gelu(x_r[...] * sc + sh)
        xn_o[...] = xn
        y = jnp.dot(xn, wl_r[...], preferred_element_type=jnp.float32) + bl_r[...]
        qa_o[...] = jnp.dot(y, wq_r[...], preferred_element_type=jnp.float32) - bp2a_r[...]
        ka = jnp.dot(y, wk_r[...], preferred_element_type=jnp.float32) + ba_r[...]
        v = jnp.dot(y, wv_r[...], preferred_element_type=jnp.float32) + bp2_r[...]
        ua = lax.bitcast_convert_type(ka, jnp.uint32) + jnp.uint32(0x8000)
        uv = lax.bitcast_convert_type(v, jnp.uint32) + jnp.uint32(0x8000)
        packed = (ua & jnp.uint32(0xFFFF0000)) | (uv >> jnp.uint32(16))
        kv_o[...] = lax.bitcast_convert_type(packed, jnp.int32)

    return pl.pallas_call(
        body,
        grid=(GRID,),
        in_specs=[
            pl.BlockSpec((B, C), lambda i: (i, 0)),
            _full((8, C)), _full((1, C)), _full((1, C)),
            _full((C, C)), _full((1, C)),
            _full((C, C)), _full((C, C)), _full((1, C)), _full((C, C)),
            _full((1, C)), _full((1, C)),
        ],
        out_specs=[
            pl.BlockSpec((B, C), lambda i: (i, 0)),
            pl.BlockSpec((B, C), lambda i: (i, 0)),
            pl.BlockSpec((B, C), lambda i: (i, 0)),
        ],
        out_shape=[
            jax.ShapeDtypeStruct((N, C), jnp.float32),
            jax.ShapeDtypeStruct((N, C), jnp.float32),
            jax.ShapeDtypeStruct((N, C), jnp.int32),
        ],
        compiler_params=pltpu.CompilerParams(
            dimension_semantics=("arbitrary",)),
    )(x, st1, g1, be1, Wl1, bl1, WqA, WkA, ba1, Wv, bp2A, bp2)


def _tc3(s0, kvn, rel16, qA, xn, Wp1p, bp1p, Wp2p, Wp2Ap, Wa2,
         Wl2, bl2, Ws1, bs1, Ws2, bs2):
    def body(kvn_r, rel_r, qa_r, xn_r, wp1_r, bp1_r, wp2_r, wp2a_r,
             wa2_r, wl2_r, bl2_r, ws1_r, bs1_r, ws2_r, bs2_r,
             s_o, st_o):
        i = pl.program_id(0)
        rel = rel_r[...].reshape(K * B, 16)
        e = jax.nn.relu(jnp.dot(rel, wp1_r[...], preferred_element_type=jnp.float32)
                        + bp1_r[...]).astype(jnp.bfloat16)
        pos = jnp.dot(e, wp2_r[...],
                      preferred_element_type=jnp.float32).reshape(K, B, C)
        posA = jnp.dot(e, wp2a_r[...], preferred_element_type=jnp.float32
                       ).astype(jnp.bfloat16).reshape(K, B, C)
        u = lax.bitcast_convert_type(kvn_r[...], jnp.uint32)
        kan = lax.bitcast_convert_type(
            u & jnp.uint32(0xFFFF0000), jnp.float32).astype(jnp.bfloat16)
        vn = lax.bitcast_convert_type(u << jnp.uint32(16), jnp.float32)
        qab = qa_r[...].astype(jnp.bfloat16)
        w1 = jax.nn.relu(kan - qab[None, :, :] + posA)
        w3 = jnp.dot(w1.reshape(K * B, C), wa2_r[...],
                     preferred_element_type=jnp.float32).reshape(K, B, C)
        m = jnp.max(w3, axis=0, keepdims=True)
        ew = jnp.exp(w3 - m)
        ssum = jnp.sum(ew, axis=0)
        z = vn + pos
        agg = jnp.sum(ew * z, axis=0) / ssum
        x2 = jax.nn.relu(
            xn_r[...] + jnp.dot(agg, wl2_r[...], preferred_element_type=jnp.float32) + bl2_r[...])
        h = jax.nn.gelu(jnp.dot(x2, ws1_r[...], preferred_element_type=jnp.float32) + bs1_r[...])
        s = jnp.dot(h, ws2_r[...], preferred_element_type=jnp.float32) + bs2_r[...]
        s_o[...] = s

        @pl.when(i == 0)
        def _():
            st_o[...] = jnp.zeros((8, C), jnp.float32)

        upd = jnp.concatenate(
            [jnp.sum(s, axis=0, keepdims=True),
             jnp.sum(s * s, axis=0, keepdims=True),
             jnp.zeros((6, C), jnp.float32)], axis=0)
        st_o[...] += upd

    return pl.pallas_call(
        body,
        grid=(SGRID,),
        in_specs=[
            pl.BlockSpec((K, B, C), lambda i: (0, i, 0)),
            pl.BlockSpec((K, B, 16), lambda i: (0, i, 0)),
            pl.BlockSpec((B, C), lambda i: (s0 * SGRID + i, 0)),
            pl.BlockSpec((B, C), lambda i: (s0 * SGRID + i, 0)),
            _full((16, 16)), _full((1, 16)),
            _full((16, C)),
            _full((16, C)),
            _full((C, C)),
            _full((C, C)), _full((1, C)),
            _full((C, C)), _full((1, C)),
            _full((C, C)), _full((1, C)),
        ],
        out_specs=[
            pl.BlockSpec((B, C), lambda i: (i, 0)),
            _full((8, C)),
        ],
        out_shape=[
            jax.ShapeDtypeStruct((SN, C), jnp.float32),
            jax.ShapeDtypeStruct((8, C), jnp.float32),
        ],
        compiler_params=pltpu.CompilerParams(
            dimension_semantics=("arbitrary",)),
    )(kvn, rel16, qA, xn, Wp1p, bp1p, Wp2p, Wp2Ap, Wa2,
      Wl2, bl2, Ws1, bs1, Ws2, bs2)


def _tc4(s, st2, g2, be2):
    B4 = 1000

    def body(s_r, st_r, g_r, be_r, o_r):
        st = st_r[...]
        m = st[0:1] * (1.0 / N)
        var = st[1:2] * (1.0 / N) - m * m
        sc = g_r[...] * lax.rsqrt(var + 1e-5)
        sh = be_r[...] - m * sc
        o_r[...] = jax.nn.gelu(s_r[...] * sc + sh)

    return pl.pallas_call(
        body,
        grid=(N // B4,),
        in_specs=[
            pl.BlockSpec((B4, C), lambda i: (i, 0)),
            _full((8, C)), _full((1, C)), _full((1, C)),
        ],
        out_specs=pl.BlockSpec((B4, C), lambda i: (i, 0)),
        out_shape=jax.ShapeDtypeStruct((N, C), jnp.float32),
    )(s, st2, g2, be2)


def kernel(coord, feat, offset, neighbor_idx, params):
    p = params
    dirs16 = jnp.zeros((16, NG), jnp.float32).at[:3].set(p['gib_dirs'].T)
    sig = jax.nn.softplus(p['gib_sigma']) + 1e-4
    sig2inv = (1.0 / (2.0 * sig * sig))[None, :]
    cvx_smT = jax.nn.softmax(p['cvx'], axis=1).T
    coord16 = jnp.zeros((N, 16), jnp.float32).at[:, :3].set(coord)
    WqA = p['Wq'] @ p['Wa1']
    WkA = p['Wk'] @ p['Wa1']
    Wp1p = jnp.zeros((16, 16), jnp.float32).at[:3, :3].set(p['Wp1']).astype(jnp.bfloat16)
    bp1p = jnp.zeros((1, 16), jnp.float32).at[0, :3].set(p['bp1'])
    Wp2p = jnp.zeros((16, C), jnp.float32).at[:3].set(p['Wp2'])
    Wp2Ap = (Wp2p @ p['Wa1']).astype(jnp.bfloat16)
    bp2A = (p['bp2'] @ p['Wa1'])[None, :]
    Wp2p = Wp2p.astype(jnp.bfloat16)

    nbr_flat = neighbor_idx.T.reshape(-1)

    rel16_s, x_s, st1_s = [], [], []
    for s in range(S):
        cn = _sc_gather(coord16, lax.slice(nbr_flat, (s * SE,), ((s + 1) * SE,)),
                        16).reshape(K, SN, 16)
        rel16, x, st1 = _tc1(
            s, coord16, cn, feat, dirs16, sig2inv, cvx_smT,
            p['W_enc'], p['b_enc'][None, :], p['W1'], p['b1'][None, :],
            p['W2'], p['b2'][None, :])
        rel16_s.append(rel16)
        x_s.append(x)
        st1_s.append(st1)
    st1 = functools.reduce(lambda a, b: a + b, st1_s)
    x = jnp.concatenate(x_s, axis=0)

    xn, qA, kv = _tc2(
        x, st1, p['g1'][None, :], p['be1'][None, :],
        p['Wl1'], p['bl1'][None, :], WqA, WkA,
        p['ba1'][None, :], p['Wv'], bp2A, p['bp2'][None, :])

    s_s, st2_s = [], []
    for s in range(S):
        kvn = _sc_gather(kv, lax.slice(nbr_flat, (s * SE,), ((s + 1) * SE,)),
                         C, jnp.int32).reshape(K, SN, C)
        ss, st2 = _tc3(
            s, kvn, rel16_s[s], qA, xn, Wp1p, bp1p, Wp2p,
            Wp2Ap, p['Wa2'].astype(jnp.bfloat16),
            p['Wl2'],
            p['bl2'][None, :], p['Ws1'], p['bs1'][None, :], p['Ws2'],
            p['bs2'][None, :])
        s_s.append(ss)
        st2_s.append(st2)
    st2 = functools.reduce(lambda a, b: a + b, st2_s)
    s = jnp.concatenate(s_s, axis=0)

    out = _tc4(s, st2, p['g2'][None, :], p['be2'][None, :])
    return (coord, out, offset)
